# baseline (speedup 1.0000x reference)
import functools

import jax
import jax.numpy as jnp
from jax import lax
from jax.experimental import pallas as pl
from jax.experimental.pallas import tpu as pltpu
from jax.experimental.pallas import tpu_sc as plsc

BOX = 80
RES = 1.0
EPS_IN = 6.5
EPS_OUT = 79.0
B = 16
A = 8000

NSLAB = 4
SLABX = BOX // NSLAB
SLAB_WORDS = SLABX * BOX * BOX
PLANE = BOX * BOX

CHUNK = 256
APAD = 8192
NCHUNK = APAD // CHUNK
VREGS_PER_CHUNK = CHUNK // 16
GROWS = 8

NW = 32
NTASK = B * NSLAB

_mesh = plsc.VectorSubcoreMesh(
    core_axis_name="c", subcore_axis_name="s", num_cores=2, num_subcores=16)


@functools.partial(
    pl.kernel,
    out_type=jax.ShapeDtypeStruct((B * BOX * BOX * BOX,), jnp.float32),
    mesh=_mesh,
    scratch_types=[
        pltpu.VMEM((SLAB_WORDS,), jnp.float32),
        pltpu.VMEM((GROWS, 128), jnp.float32),
        pltpu.VMEM((GROWS, 128), jnp.float32),
        pltpu.VMEM((16,), jnp.int32),
        pltpu.SemaphoreType.DMA,
        pltpu.SemaphoreType.DMA,
    ],
    compiler_params=pltpu.CompilerParams(needs_layout_passes=False),
)
def _splat_eps(cw_hbm, num_hbm, out_hbm,
               slab, cbufA, cbufB, nbuf, semA, semB):
    wid = lax.axis_index("s") * 2 + lax.axis_index("c")
    pltpu.sync_copy(num_hbm, nbuf)

    iota = lax.iota(jnp.int32, 16)
    zeros = jnp.zeros((16,), jnp.float32)
    f_scale = jnp.full((16,), EPS_OUT - EPS_IN, jnp.float32)
    f_off = jnp.full((16,), EPS_IN, jnp.float32)
    ones = jnp.ones((16,), jnp.float32)

    def run_task(task):
        b = task // NSLAB
        slab_i = task - b * NSLAB
        x0 = slab_i * SLABX
        na = plsc.load_gather(nbuf, [jnp.broadcast_to(b, (16,)).astype(jnp.int32)])

        def zero_body(i, _):
            base = i * 512
            for k in range(32):
                slab[pl.ds(base + k * 16, 16)] = zeros
            return 0
        lax.fori_loop(0, SLAB_WORDS // 512, zero_body, 0)

        def c_src(ci):
            return cw_hbm.at[pl.ds(b * (NCHUNK * GROWS) + ci * GROWS, GROWS), :]

        def fire(ci, cb, sem):
            pltpu.async_copy(c_src(ci), cb, sem)

        def wait(ci, cb, sem):
            pltpu.make_async_copy(c_src(ci), cb, sem).wait()

        def process(ci, cb):
            def vreg_body(j, _):
                al = iota + j * 16
                aid = al + ci * CHUNK
                al3 = al * 3
                x = plsc.load_gather(cb, [al3 >> 7, al3 & 127])
                a1 = al3 + 1
                y = plsc.load_gather(cb, [a1 >> 7, a1 & 127])
                a2 = al3 + 2
                z = plsc.load_gather(cb, [a2 >> 7, a2 & 127])
                w = plsc.load_gather(cb, [(al >> 7) + 6, al & 127])

                ix = x.astype(jnp.int32)
                iy = y.astype(jnp.int32)
                iz = z.astype(jnp.int32)
                frx = x - ix.astype(jnp.float32)
                fry = y - iy.astype(jnp.float32)
                frz = z - iz.astype(jnp.float32)

                am = aid < na
                m0 = am & (ix >= x0) & (ix < x0 + SLABX)
                ixp = ix + 1
                m1 = am & (ixp >= x0) & (ixp < x0 + SLABX)

                s0 = jnp.clip(ix - x0, 0, SLABX - 1) * PLANE
                s1 = jnp.clip(ixp - x0, 0, SLABX - 1) * PLANE
                yb0 = iy * BOX
                yb1 = yb0 + BOX

                t0 = w * (ones - frx)
                t1 = w * frx
                wy0 = ones - fry
                wz0 = ones - frz
                p00 = t0 * wy0
                p01 = t0 * fry
                p10 = t1 * wy0
                p11 = t1 * fry

                i00 = s0 + yb0 + iz
                i01 = s0 + yb1 + iz
                i10 = s1 + yb0 + iz
                i11 = s1 + yb1 + iz
                plsc.addupdate_scatter(slab, [i00], p00 * wz0, mask=m0)
                plsc.addupdate_scatter(slab, [i00 + 1], p00 * frz, mask=m0)
                plsc.addupdate_scatter(slab, [i01], p01 * wz0, mask=m0)
                plsc.addupdate_scatter(slab, [i01 + 1], p01 * frz, mask=m0)
                plsc.addupdate_scatter(slab, [i10], p10 * wz0, mask=m1)
                plsc.addupdate_scatter(slab, [i10 + 1], p10 * frz, mask=m1)
                plsc.addupdate_scatter(slab, [i11], p11 * wz0, mask=m1)
                plsc.addupdate_scatter(slab, [i11 + 1], p11 * frz, mask=m1)
                return 0

            lax.fori_loop(0, VREGS_PER_CHUNK, vreg_body, 0)

        fire(0, cbufA, semA)

        def pair_body(k, _):
            ci0 = 2 * k
            ci1 = ci0 + 1
            fire(ci1, cbufB, semB)
            wait(ci0, cbufA, semA)
            process(ci0, cbufA)

            @pl.when(k < NCHUNK // 2 - 1)
            def _():
                fire(ci0 + 2, cbufA, semA)

            wait(ci1, cbufB, semB)
            process(ci1, cbufB)
            return 0

        lax.fori_loop(0, NCHUNK // 2, pair_body, 0)

        def eps_body(i, _):
            base = i * 512
            for k in range(32):
                off = base + k * 16
                v = slab[pl.ds(off, 16)]
                slab[pl.ds(off, 16)] = jnp.exp(-v) * f_scale + f_off
            return 0
        lax.fori_loop(0, SLAB_WORDS // 512, eps_body, 0)

        pltpu.sync_copy(
            slab,
            out_hbm.at[pl.ds(b * (BOX * BOX * BOX) + x0 * PLANE, SLAB_WORDS)])

    run_task(wid)
    run_task(wid + NW)


def _stage_body(cref, wref, out_ref):
    c3 = cref[...].reshape(8, NCHUNK, 6, 128)
    w3 = wref[...].reshape(8, NCHUNK, 2, 128)
    packed = jnp.concatenate([c3, w3], axis=2)
    out_ref[...] = packed.reshape(8 * NCHUNK * GROWS, 128)


def _stage(cpad, wpad):
    return pl.pallas_call(
        _stage_body,
        grid=(B // 8,),
        in_specs=[
            pl.BlockSpec((8, APAD * 3), lambda g: (g, 0)),
            pl.BlockSpec((8, APAD), lambda g: (g, 0)),
        ],
        out_specs=pl.BlockSpec((8 * NCHUNK * GROWS, 128), lambda g: (g, 0)),
        out_shape=jax.ShapeDtypeStruct((B * NCHUNK * GROWS, 128), jnp.float32),
    )(cpad, wpad)


def kernel(coords, assigned_params, num_atoms):
    cpad = jnp.pad(coords, ((0, 0), (0, APAD * 3 - A * 3)))
    wpad = jnp.pad(assigned_params[:, :, 1], ((0, 0), (0, APAD - A)))
    cw = _stage(cpad, wpad)
    out = _splat_eps(cw, num_atoms)
    return out.reshape(B, BOX, BOX, BOX)
